# baseline (speedup 1.0000x reference)
import functools

import jax
import jax.numpy as jnp
from jax import lax
from jax.experimental import pallas as pl
from jax.experimental.pallas import tpu as pltpu
from jax.experimental.pallas import tpu_sc as plsc

NUM_EMB = 1_000_000
D = 64
B = 16384

_info = plsc.get_sparse_core_info()
_NC, _NS, _L = _info.num_cores, _info.num_subcores, _info.num_lanes
_NW = _NS
_BH = B // 2
_BPW = _BH // _NW
_G = _BPW // _L

_mesh1 = plsc.VectorSubcoreMesh(
    core_axis_name="c", subcore_axis_name="s", num_cores=1
)


@functools.partial(
    pl.kernel,
    mesh=_mesh1,
    out_type=jax.ShapeDtypeStruct((_BH, D), jnp.float32),
    scratch_types=[
        pltpu.VMEM((_BPW,), jnp.int32),
        pltpu.VMEM((_L, 8, D), jnp.float32),
        pltpu.VMEM((_BPW, D), jnp.float32),
        pltpu.VMEM((D,), jnp.float32),
        pltpu.SemaphoreType.DMA,
    ],
    compiler_params=pltpu.CompilerParams(
        use_tc_tiling_on_sc=True,
        needs_layout_passes=False,
    ),
)
def _kb_half(idx_hbm, t_hbm, rel_hbm, out_hbm,
             idx_v, tile_v, rows_v, rel_v, sem):
    wid = lax.axis_index("s")
    base = wid * _BPW

    pltpu.sync_copy(idx_hbm.at[pl.ds(base, _BPW)], idx_v)
    pltpu.sync_copy(rel_hbm, rel_v)

    rel_c = [rel_v[pl.ds(c * _L, _L)] for c in range(D // _L)]
    lane = lax.iota(jnp.int32, _L)

    def grp_body(g, carry):
        v = idx_v[pl.ds(g * _L, _L)]
        vt = lax.shift_left(lax.shift_right_logical(v, 3), 3)
        sub = lax.bitwise_and(v, 7)
        for j in range(_L):
            rt = lax.reduce_max(
                jnp.where(lane == j, vt, jnp.int32(0)), axes=(0,)
            )
            rt = pl.multiple_of(rt, 8)
            pltpu.async_copy(
                t_hbm.at[pl.ds(rt, 8)], tile_v.at[j], sem
            )
        for j in range(_L):
            pltpu.make_async_copy(
                t_hbm.at[pl.ds(0, 8)], tile_v.at[j], sem
            ).wait()
        for j in range(_L):
            j_vec = lax.broadcast(jnp.int32(j), (_L,))
            s_scalar = lax.reduce_max(
                jnp.where(lane == j, sub, jnp.int32(0)), axes=(0,)
            )
            s_vec = lax.broadcast(s_scalar, (_L,))
            for c in range(D // _L):
                col = (c * _L) + lane
                vals = plsc.load_gather(tile_v, [j_vec, s_vec, col])
                rows_v[g * _L + j, pl.ds(c * _L, _L)] = vals + rel_c[c]
        return carry

    lax.fori_loop(0, _G, grp_body, 0)

    pltpu.sync_copy(rows_v, out_hbm.at[pl.ds(base, _BPW)])


def kernel(entity_idx, entity_table, relation_embedding):
    idx = entity_idx.astype(jnp.int32)
    out0 = _kb_half(idx[:_BH], entity_table, relation_embedding)
    out1 = _kb_half(idx[_BH:], entity_table, relation_embedding)
    return jnp.concatenate([out0, out1], axis=0)


# speedup vs baseline: 1.1520x; 1.1520x over previous
import functools

import jax
import jax.numpy as jnp
from jax import lax
from jax.experimental import pallas as pl
from jax.experimental.pallas import tpu as pltpu
from jax.experimental.pallas import tpu_sc as plsc

NUM_EMB = 1_000_000
D = 64
B = 16384

_info = plsc.get_sparse_core_info()
_NC, _NS, _L = _info.num_cores, _info.num_subcores, _info.num_lanes
_NW = _NC * _NS
_BPW = B // _NW
_G = _BPW // _L

_mesh = plsc.VectorSubcoreMesh(core_axis_name="c", subcore_axis_name="s")


@functools.partial(
    pl.kernel,
    mesh=_mesh,
    out_type=jax.ShapeDtypeStruct((D, B), jnp.float32),
    scratch_types=[
        pltpu.VMEM((_BPW,), jnp.int32),
        pltpu.VMEM((_L, 8, D), jnp.float32),
        pltpu.VMEM((D, _BPW), jnp.float32),
        pltpu.VMEM((D,), jnp.float32),
        pltpu.SemaphoreType.DMA,
    ],
    compiler_params=pltpu.CompilerParams(
        use_tc_tiling_on_sc=True,
        needs_layout_passes=False,
    ),
)
def _kb_lookup(idx_hbm, t_hbm, rel_hbm, outT_hbm,
               idx_v, tile_v, panT_v, rel_v, sem):
    wid = lax.axis_index("s") * _NC + lax.axis_index("c")
    base = wid * _BPW

    pltpu.sync_copy(idx_hbm.at[pl.ds(base, _BPW)], idx_v)
    pltpu.sync_copy(rel_hbm, rel_v)

    rel_c = [rel_v[pl.ds(c * _L, _L)] for c in range(D // _L)]
    lane = lax.iota(jnp.int32, _L)

    def grp_body(g, carry):
        v = idx_v[pl.ds(g * _L, _L)]
        vt = lax.shift_left(lax.shift_right_logical(v, 3), 3)
        sub = lax.bitwise_and(v, 7)
        for j in range(_L):
            rt = lax.reduce_max(
                jnp.where(lane == j, vt, jnp.int32(0)), axes=(0,)
            )
            rt = pl.multiple_of(rt, 8)
            pltpu.async_copy(
                t_hbm.at[pl.ds(rt, 8)], tile_v.at[j], sem
            )
        for j in range(_L):
            pltpu.make_async_copy(
                t_hbm.at[pl.ds(0, 8)], tile_v.at[j], sem
            ).wait()
        for j in range(_L):
            j_vec = lax.broadcast(jnp.int32(j), (_L,))
            s_scalar = lax.reduce_max(
                jnp.where(lane == j, sub, jnp.int32(0)), axes=(0,)
            )
            s_vec = lax.broadcast(s_scalar, (_L,))
            col_out = lax.broadcast(jnp.int32(g * _L + j), (_L,))
            for c in range(D // _L):
                col = (c * _L) + lane
                vals = plsc.load_gather(tile_v, [j_vec, s_vec, col])
                plsc.store_scatter(
                    panT_v, [col, col_out], vals + rel_c[c]
                )
        return carry

    lax.fori_loop(0, _G, grp_body, 0)

    pltpu.sync_copy(panT_v, outT_hbm.at[:, pl.ds(base, _BPW)])


def kernel(entity_idx, entity_table, relation_embedding):
    outT = _kb_lookup(
        entity_idx.astype(jnp.int32), entity_table, relation_embedding
    )
    return outT.T


# speedup vs baseline: 1.1602x; 1.0071x over previous
import functools

import jax
import jax.numpy as jnp
from jax import lax
from jax.experimental import pallas as pl
from jax.experimental.pallas import tpu as pltpu
from jax.experimental.pallas import tpu_sc as plsc

NUM_EMB = 1_000_000
D = 64
B = 16384

_info = plsc.get_sparse_core_info()
_NC, _NS, _L = _info.num_cores, _info.num_subcores, _info.num_lanes
_NW = _NC * _NS
_BPW = B // _NW
_G = _BPW // _L

_mesh = plsc.VectorSubcoreMesh(core_axis_name="c", subcore_axis_name="s")


@functools.partial(
    pl.kernel,
    mesh=_mesh,
    out_type=jax.ShapeDtypeStruct((B, D), jnp.float32),
    scratch_types=[
        pltpu.VMEM((_BPW,), jnp.int32),
        pltpu.VMEM((_L, 8, D), jnp.float32),
        pltpu.VMEM((_BPW, D), jnp.float32),
        pltpu.VMEM((D,), jnp.float32),
        pltpu.SemaphoreType.DMA,
    ],
    compiler_params=pltpu.CompilerParams(
        use_tc_tiling_on_sc=True,
        needs_layout_passes=False,
        skip_device_barrier=True,
    ),
)
def _kb_lookup(idx_hbm, t_hbm, rel_hbm, out_hbm,
               idx_v, tile_v, rows_v, rel_v, sem):
    wid = lax.axis_index("s") * _NC + lax.axis_index("c")
    base = wid * _BPW

    pltpu.sync_copy(idx_hbm.at[pl.ds(base, _BPW)], idx_v)
    pltpu.sync_copy(rel_hbm, rel_v)

    rel_c = [rel_v[pl.ds(c * _L, _L)] for c in range(D // _L)]
    lane = lax.iota(jnp.int32, _L)

    def grp_body(g, carry):
        v = idx_v[pl.ds(g * _L, _L)]
        vt = lax.shift_left(lax.shift_right_logical(v, 3), 3)
        sub = lax.bitwise_and(v, 7)
        for j in range(_L):
            rt = lax.reduce_max(
                jnp.where(lane == j, vt, jnp.int32(0)), axes=(0,)
            )
            rt = pl.multiple_of(rt, 8)
            pltpu.async_copy(
                t_hbm.at[pl.ds(rt, 8)], tile_v.at[j], sem
            )
        for j in range(_L):
            pltpu.make_async_copy(
                t_hbm.at[pl.ds(0, 8)], tile_v.at[j], sem
            ).wait()
        for j in range(_L):
            j_vec = lax.broadcast(jnp.int32(j), (_L,))
            s_scalar = lax.reduce_max(
                jnp.where(lane == j, sub, jnp.int32(0)), axes=(0,)
            )
            s_vec = lax.broadcast(s_scalar, (_L,))
            for c in range(D // _L):
                col = (c * _L) + lane
                vals = plsc.load_gather(tile_v, [j_vec, s_vec, col])
                rows_v[g * _L + j, pl.ds(c * _L, _L)] = vals + rel_c[c]
        return carry

    lax.fori_loop(0, _G, grp_body, 0)

    pltpu.sync_copy(rows_v, out_hbm.at[pl.ds(base, _BPW)])


def kernel(entity_idx, entity_table, relation_embedding):
    return _kb_lookup(
        entity_idx.astype(jnp.int32), entity_table, relation_embedding
    )


# speedup vs baseline: 1.2214x; 1.0528x over previous
import functools

import jax
import jax.numpy as jnp
from jax import lax
from jax.experimental import pallas as pl
from jax.experimental.pallas import tpu as pltpu
from jax.experimental.pallas import tpu_sc as plsc

NUM_EMB = 1_000_000
D = 64
B = 16384

_info = plsc.get_sparse_core_info()
_NC, _NS, _L = _info.num_cores, _info.num_subcores, _info.num_lanes
_NW = _NC * _NS
_BPW = B // _NW
_G = _BPW // _L

_mesh = plsc.VectorSubcoreMesh(core_axis_name="c", subcore_axis_name="s")


@functools.partial(
    pl.kernel,
    mesh=_mesh,
    out_type=jax.ShapeDtypeStruct((B, D), jnp.float32),
    scratch_types=[
        pltpu.VMEM((_BPW,), jnp.int32),
        pltpu.VMEM((2, _L, 8, D), jnp.float32),
        pltpu.VMEM((_BPW, D), jnp.float32),
        pltpu.VMEM((D,), jnp.float32),
        pltpu.SemaphoreType.DMA((2,)),
    ],
    compiler_params=pltpu.CompilerParams(
        use_tc_tiling_on_sc=True,
        needs_layout_passes=False,
        skip_device_barrier=True,
    ),
)
def _kb_lookup(idx_hbm, t_hbm, rel_hbm, out_hbm,
               idx_v, tile_v, rows_v, rel_v, sem):
    wid = lax.axis_index("s") * _NC + lax.axis_index("c")
    base = wid * _BPW

    pltpu.sync_copy(idx_hbm.at[pl.ds(base, _BPW)], idx_v)
    pltpu.sync_copy(rel_hbm, rel_v)

    rel_c = [rel_v[pl.ds(c * _L, _L)] for c in range(D // _L)]
    lane = lax.iota(jnp.int32, _L)

    def issue_group(g, slot):
        v = idx_v[pl.ds(g * _L, _L)]
        vt = lax.shift_left(lax.shift_right_logical(v, 3), 3)
        for j in range(_L):
            rt = lax.reduce_max(
                jnp.where(lane == j, vt, jnp.int32(0)), axes=(0,)
            )
            rt = pl.multiple_of(rt, 8)
            pltpu.async_copy(
                t_hbm.at[pl.ds(rt, 8)], tile_v.at[slot, j], sem.at[slot]
            )

    issue_group(0, 0)

    def grp_body(g, carry):
        slot = lax.rem(g, 2)

        @pl.when(g + 1 < _G)
        def _():
            issue_group(g + 1, lax.rem(g + 1, 2))

        for j in range(_L):
            pltpu.make_async_copy(
                t_hbm.at[pl.ds(0, 8)], tile_v.at[slot, j], sem.at[slot]
            ).wait()

        v = idx_v[pl.ds(g * _L, _L)]
        sub = lax.bitwise_and(v, 7)
        slot_vec = lax.broadcast(slot, (_L,))
        for j in range(_L):
            j_vec = lax.broadcast(jnp.int32(j), (_L,))
            s_scalar = lax.reduce_max(
                jnp.where(lane == j, sub, jnp.int32(0)), axes=(0,)
            )
            s_vec = lax.broadcast(s_scalar, (_L,))
            for c in range(D // _L):
                col = (c * _L) + lane
                vals = plsc.load_gather(
                    tile_v, [slot_vec, j_vec, s_vec, col]
                )
                rows_v[g * _L + j, pl.ds(c * _L, _L)] = vals + rel_c[c]
        return carry

    lax.fori_loop(0, _G, grp_body, 0)

    pltpu.sync_copy(rows_v, out_hbm.at[pl.ds(base, _BPW)])


def kernel(entity_idx, entity_table, relation_embedding):
    return _kb_lookup(
        entity_idx.astype(jnp.int32), entity_table, relation_embedding
    )
